# baseline (speedup 1.0000x reference)
import functools

import jax
import jax.numpy as jnp
from jax import lax
from jax.experimental import pallas as pl
from jax.experimental.pallas import tpu as pltpu
from jax.experimental.pallas import tpu_sc as plsc

B = 4096
NUM = 13
NCAT = 26
CARD = 1000
D = 128
NTOK = NUM + NCAT
TBL = NCAT * (CARD + 1)

NC = 2
NS = 16
NW = NC * NS
BPW = B // NW
IDXW = 32
OC = 8
NCHUNK = BPW // OC
NPAIR = NCHUNK // 2


def _tok_body(xnum_hbm, xcat_hbm, w_hbm, b_hbm, emb_hbm, out_hbm,
              xnum_v, idx_v, w_v, bias_v, tok0, tok1, gsem, wsem0, wsem1):
    wid = lax.axis_index("s") * NC + lax.axis_index("c")
    base_b = wid * BPW

    pltpu.sync_copy(xnum_hbm.at[pl.ds(base_b * NUM, BPW * NUM)],
                    xnum_v.at[pl.ds(0, BPW * NUM)])
    pltpu.sync_copy(xcat_hbm.at[pl.ds(base_b, BPW), :], idx_v)
    pltpu.sync_copy(w_hbm, w_v)
    pltpu.sync_copy(b_hbm, bias_v)

    lane = lax.iota(jnp.int32, 16)
    for v in range(IDXW // 16):
        offs = ((lane + v * 16) % NCAT) * (CARD + 1)
        for r in range(BPW):
            sl = pl.ds(v * 16, 16)
            idx_v[r, sl] = idx_v[r, sl] + offs

    def compute_num(j, tok, blo, bhi):
        xrs = [xnum_v[pl.ds((j * OC + b) * NUM, 16)] for b in range(blo, bhi)]
        for f in range(NUM):
            wv = [w_v[f, pl.ds(v * 16, 16)] for v in range(D // 16)]
            bv = [bias_v[f, pl.ds(v * 16, 16)] for v in range(D // 16)]
            for i, b in enumerate(range(blo, bhi)):
                xv = jnp.full((16,), xrs[i][f], dtype=jnp.float32)
                for v in range(D // 16):
                    tok[b, f, pl.ds(v * 16, 16)] = xv * wv[v] + bv[v]

    def compute_chunk(j, tok, wsem):
        gs = []
        for b in range(OC):
            gs.append(pltpu.async_copy(
                emb_hbm.at[idx_v.at[j * OC + b, pl.ds(0, NCAT)]],
                tok.at[b, pl.ds(NUM, NCAT), :], gsem))

        h = OC // 2
        compute_num(j, tok, 0, h)
        for g in gs[:h]:
            g.wait()
        pltpu.async_copy(
            tok.at[pl.ds(0, h)],
            out_hbm.at[pl.ds(base_b + j * OC, h), :, :], wsem)
        compute_num(j, tok, h, OC)
        for g in gs[h:]:
            g.wait()
        pltpu.async_copy(
            tok.at[pl.ds(h, h)],
            out_hbm.at[pl.ds(base_b + j * OC + h, h), :, :], wsem)

    def drain_write(tok, wsem):
        pltpu.make_async_copy(
            out_hbm.at[pl.ds(base_b, OC), :, :], tok, wsem).wait()

    def pair(t, carry):
        @pl.when(t >= 1)
        def _():
            drain_write(tok0, wsem0)
        compute_chunk(2 * t, tok0, wsem0)

        @pl.when(t >= 1)
        def _():
            drain_write(tok1, wsem1)
        compute_chunk(2 * t + 1, tok1, wsem1)
        return carry

    lax.fori_loop(0, NPAIR, pair, 0)
    drain_write(tok0, wsem0)
    drain_write(tok1, wsem1)


_tok_kernel = functools.partial(
    pl.kernel,
    out_type=jax.ShapeDtypeStruct((B, NTOK, D), jnp.float32),
    mesh=plsc.VectorSubcoreMesh(core_axis_name="c", subcore_axis_name="s"),
    scratch_types=[
        pltpu.VMEM((BPW * NUM + 16,), jnp.float32),
        pltpu.VMEM((BPW, IDXW), jnp.int32),
        pltpu.VMEM((NUM, D), jnp.float32),
        pltpu.VMEM((NUM, D), jnp.float32),
        pltpu.VMEM((OC, NTOK, D), jnp.float32),
        pltpu.VMEM((OC, NTOK, D), jnp.float32),
        pltpu.SemaphoreType.DMA,
        pltpu.SemaphoreType.DMA,
        pltpu.SemaphoreType.DMA,
    ],
)(_tok_body)


@jax.jit
def kernel(x_num, x_cat, num_weight, num_bias, cat_emb):
    xcat_pad = jnp.pad(x_cat, ((0, 0), (0, IDXW - NCAT)))
    return _tok_kernel(
        x_num.reshape(-1),
        xcat_pad,
        num_weight,
        num_bias,
        cat_emb.reshape(TBL, D),
    )
